# baseline (speedup 1.0000x reference)
import functools

import jax
import jax.numpy as jnp
from jax import lax
from jax.experimental import pallas as pl
from jax.experimental.pallas import tpu as pltpu
from jax.experimental.pallas import tpu_sc as plsc

B = 16384
DIM = 32
NW = 32
BPW = B // NW
NCH = 4
CH = BPW // NCH
NBLK = CH // 16

_mesh = plsc.VectorSubcoreMesh(core_axis_name="c", subcore_axis_name="s")


@functools.partial(
    pl.kernel,
    mesh=_mesh,
    compiler_params=pltpu.CompilerParams(
        needs_layout_passes=False, use_tc_tiling_on_sc=False),
    out_type=jax.ShapeDtypeStruct((B,), jnp.float32),
    scratch_types=[
        pltpu.VMEM((NCH, CH), jnp.int32),
        pltpu.VMEM((NCH, CH), jnp.int32),
        pltpu.VMEM((NCH, CH), jnp.int32),
        pltpu.VMEM((BPW, DIM), jnp.float32),
        pltpu.VMEM((BPW, DIM), jnp.float32),
        pltpu.VMEM((BPW, DIM), jnp.float32),
        pltpu.VMEM((BPW, DIM), jnp.float32),
        pltpu.VMEM((256,), jnp.float32),
        pltpu.VMEM((BPW,), jnp.float32),
        pltpu.SemaphoreType.DMA,
    ],
)
def _mure_sc(u_idx_hbm, v_idx_hbm, r_idx_hbm, e_hbm, wu_hbm, rv_hbm, out_hbm,
             ui_v, vi_v, ri_v, u_v, v_v, ru_v, rw_v, tmp_v, out_v, sem):
    wid = lax.axis_index("s") * 2 + lax.axis_index("c")
    base = wid * BPW

    pltpu.sync_copy(u_idx_hbm.at[wid], ui_v)
    pltpu.sync_copy(v_idx_hbm.at[wid], vi_v)
    pltpu.sync_copy(r_idx_hbm.at[wid], ri_v)

    copies = []
    for k in range(NCH):
        copies.append(pltpu.async_copy(
            e_hbm.at[ui_v.at[k]], u_v.at[pl.ds(k * CH, CH)], sem))
        copies.append(pltpu.async_copy(
            e_hbm.at[vi_v.at[k]], v_v.at[pl.ds(k * CH, CH)], sem))
        copies.append(pltpu.async_copy(
            wu_hbm.at[ri_v.at[k]], ru_v.at[pl.ds(k * CH, CH)], sem))
        copies.append(pltpu.async_copy(
            rv_hbm.at[ri_v.at[k]], rw_v.at[pl.ds(k * CH, CH)], sem))
    for c in copies:
        c.wait()

    lane = lax.iota(jnp.int32, 16)
    lane16 = lane * 16

    def blk(b, carry):
        for e in range(16):
            i = b * 16 + e
            d0 = (u_v[i, pl.ds(0, 16)] * ru_v[i, pl.ds(0, 16)]
                  - v_v[i, pl.ds(0, 16)] - rw_v[i, pl.ds(0, 16)])
            d1 = (u_v[i, pl.ds(16, 16)] * ru_v[i, pl.ds(16, 16)]
                  - v_v[i, pl.ds(16, 16)] - rw_v[i, pl.ds(16, 16)])
            tmp_v[pl.ds(e * 16, 16)] = d0 * d0 + d1 * d1
        res = jnp.zeros((16,), dtype=jnp.float32)
        for l in range(16):
            res = res + plsc.load_gather(tmp_v, [lane16 + l])
        out_v[pl.ds(b * 16, 16)] = -res
        return carry

    lax.fori_loop(0, BPW // 16, blk, 0)

    pltpu.sync_copy(out_v, out_hbm.at[pl.ds(base, BPW)])


def kernel(u_idx, r_idx, v_idx, E, Wu, rv, bs, bo):
    del bs, bo
    ui = u_idx.astype(jnp.int32).reshape(NW, NCH, CH)
    vi = v_idx.astype(jnp.int32).reshape(NW, NCH, CH)
    ri = r_idx.astype(jnp.int32).reshape(NW, NCH, CH)
    return _mure_sc(ui, vi, ri, E, Wu, rv)


# speedup vs baseline: 1.5096x; 1.5096x over previous
import functools

import jax
import jax.numpy as jnp
from jax import lax
from jax.experimental import pallas as pl
from jax.experimental.pallas import tpu as pltpu
from jax.experimental.pallas import tpu_sc as plsc

B = 16384
DIM = 32
NW = 32
BPW = B // NW
NBLK = BPW // 16

_mesh = plsc.VectorSubcoreMesh(core_axis_name="c", subcore_axis_name="s")


@functools.partial(
    pl.kernel,
    mesh=_mesh,
    compiler_params=pltpu.CompilerParams(needs_layout_passes=False),
    out_type=jax.ShapeDtypeStruct((B,), jnp.float32),
    scratch_types=[
        pltpu.VMEM((BPW,), jnp.int32),
        pltpu.VMEM((BPW,), jnp.int32),
        pltpu.VMEM((BPW,), jnp.int32),
        pltpu.VMEM((128, 128), jnp.float32),
        pltpu.VMEM((128, 128), jnp.float32),
        pltpu.VMEM((128, 128), jnp.float32),
        pltpu.VMEM((128, 128), jnp.float32),
        pltpu.VMEM((B,), jnp.int32),
        pltpu.VMEM((BPW,), jnp.float32),
        pltpu.SemaphoreType.DMA,
    ],
)
def _mure_sc(u_idx_hbm, v_idx_hbm, r_idx_hbm, e_hbm, wu_hbm, rv_hbm, out_hbm,
             ui_v, vi_v, ri_v, u_v, v_v, ru_v, rw_v, drain_v, out_v, sem):
    wid = lax.axis_index("s") * 2 + lax.axis_index("c")
    base = wid * BPW

    pltpu.sync_copy(u_idx_hbm.at[pl.ds(base, BPW)], ui_v)
    pltpu.sync_copy(v_idx_hbm.at[pl.ds(base, BPW)], vi_v)
    pltpu.sync_copy(r_idx_hbm.at[pl.ds(base, BPW)], ri_v)

    def issue(b, carry):
        uvec = ui_v[pl.ds(b * 16, 16)]
        vvec = vi_v[pl.ds(b * 16, 16)]
        rvec = ri_v[pl.ds(b * 16, 16)]
        c = b // 8
        for e in range(16):
            p = (b % 8) * 16 + e
            col = pl.ds(c * DIM, DIM)
            pltpu.async_copy(e_hbm.at[uvec[e]], u_v.at[p, col], sem)
            pltpu.async_copy(e_hbm.at[vvec[e]], v_v.at[p, col], sem)
            pltpu.async_copy(wu_hbm.at[rvec[e]], ru_v.at[p, col], sem)
            pltpu.async_copy(rv_hbm.at[rvec[e]], rw_v.at[p, col], sem)
        return carry

    lax.fori_loop(0, NBLK, issue, 0)

    for _ in range(4):
        pltpu.make_async_copy(u_idx_hbm, drain_v, sem).wait()

    lane = lax.iota(jnp.int32, 16)

    def blk(b, carry):
        p = (b % 8) * 16 + lane
        col0 = jnp.broadcast_to((b // 8) * DIM, (16,)).astype(jnp.int32)
        acc = jnp.zeros((16,), dtype=jnp.float32)
        for j in range(DIM):
            cj = col0 + j
            uu = plsc.load_gather(u_v, [p, cj])
            vv = plsc.load_gather(v_v, [p, cj])
            ru = plsc.load_gather(ru_v, [p, cj])
            rw = plsc.load_gather(rw_v, [p, cj])
            d = uu * ru - vv - rw
            acc = acc + d * d
        out_v[pl.ds(b * 16, 16)] = -acc
        return carry

    lax.fori_loop(0, NBLK, blk, 0)

    pltpu.sync_copy(out_v, out_hbm.at[pl.ds(base, BPW)])


def kernel(u_idx, r_idx, v_idx, E, Wu, rv, bs, bo):
    del bs, bo
    return _mure_sc(u_idx.astype(jnp.int32), v_idx.astype(jnp.int32),
                    r_idx.astype(jnp.int32), E, Wu, rv)
